# Initial kernel scaffold; baseline (speedup 1.0000x reference)
import jax
import jax.numpy as jnp
from jax.experimental import pallas as pl


def kernel(x, ln1_w, ln2_w, W_kvd, W_qd, W_ku, W_qu, W_vu, W_rk, W_rq, W_o, sh_gate, sh_up, sh_down, rt_gate, rt_up, rt_down, W_router, routing_bias):
    raise NotImplementedError("write your pallas kernel here")


# speedup vs baseline: 1.0503x; 1.0503x over previous
import jax
import jax.numpy as jnp
from jax.experimental import pallas as pl

B, T, H = 2, 2048, 1024
L = H // 4
I = int(H * 2.0)
NS = 2
NR = 8 - NS
NE = NR + NS
TOPK = 2
BASE = 10000.0
SCALE = 1.0
EPS = 1e-5
N = B * T

BQ = 256
BR = 256
RL = 128

_f32 = jnp.float32
_bf16 = jnp.bfloat16


def _ln(x, w):
    mu = jnp.mean(x, axis=1, keepdims=True)
    xc = x - mu
    var = jnp.mean(xc * xc, axis=1, keepdims=True)
    return xc * jax.lax.rsqrt(var + EPS) * w


def _prep_kernel(x_ref, ln1w_ref, wqd_ref, wrq_ref, wrk_ref, wkvd_ref,
                 wvu_ref, cos_ref, sin_ref, q_ref, k_ref, v_ref):
    x = x_ref[...]
    xb = _ln(x, ln1w_ref[...]).astype(_bf16)
    ql = jnp.dot(xb, wqd_ref[...], preferred_element_type=_f32).astype(_bf16)
    qr = jnp.dot(ql, wrq_ref[...], preferred_element_type=_f32)
    kr = jnp.dot(xb, wrk_ref[...], preferred_element_type=_f32)
    kv = jnp.dot(xb, wkvd_ref[...], preferred_element_type=_f32).astype(_bf16)
    v = jnp.dot(kv, wvu_ref[...], preferred_element_type=_f32)
    cos = cos_ref[...]
    sin = sin_ref[...]

    def rope(t):
        t1 = t[:, :H // 2]
        t2 = t[:, H // 2:]
        rot = jnp.concatenate([-t2, t1], axis=1)
        return t * cos + rot * sin

    q_ref[...] = rope(qr).astype(_bf16)
    k_ref[...] = rope(kr).astype(_bf16)
    v_ref[...] = v.astype(_bf16)


def _attn_kernel(q_ref, k_ref, v_ref, o_ref):
    qi = pl.program_id(1)
    q = q_ref[0]
    k = k_ref[0]
    s = jax.lax.dot_general(q, k, (((1,), (1,)), ((), ())),
                            preferred_element_type=_f32) * (1.0 / 32.0)
    row = qi * BQ + jax.lax.broadcasted_iota(jnp.int32, (BQ, T), 0)
    col = jax.lax.broadcasted_iota(jnp.int32, (BQ, T), 1)
    s = jnp.where(row >= col, s, -1e30)
    m = jnp.max(s, axis=1, keepdims=True)
    p = jnp.exp(s - m)
    p = p / jnp.sum(p, axis=1, keepdims=True)
    o_ref[0] = jnp.dot(p.astype(_bf16), v_ref[0],
                       preferred_element_type=_f32).astype(_bf16)


def _post_kernel(y_ref, x_ref, wo_ref, ln2w_ref, wrt_ref, bias_ref,
                 h_ref, xn2_ref, logits_ref):
    h = x_ref[...] + jnp.dot(y_ref[...], wo_ref[...],
                             preferred_element_type=_f32)
    h_ref[...] = h
    xn2 = _ln(h, ln2w_ref[...])
    xn2_ref[...] = xn2.astype(_bf16)
    logits_ref[...] = jnp.dot(xn2, wrt_ref[...],
                              preferred_element_type=_f32) + bias_ref[...]


def _moe_kernel(xn2_ref, h_ref, logits_ref, gate_ref, up_ref, down_ref,
                out_ref):
    e = pl.program_id(0)
    r = pl.program_id(1)
    xb = xn2_ref[...]
    a = jnp.dot(xb, gate_ref[0], preferred_element_type=_f32)
    b = jnp.dot(xb, up_ref[0], preferred_element_type=_f32)
    h1 = (a * jax.nn.sigmoid(a) * b).astype(_bf16)
    contrib = jnp.dot(h1, down_ref[0], preferred_element_type=_f32)

    logits = logits_ref[...]
    probs = jax.nn.sigmoid(logits)
    lane = jax.lax.broadcasted_iota(jnp.int32, (BR, RL), 1)
    pe = jnp.sum(jnp.where(lane == e, probs, 0.0), axis=1, keepdims=True)
    gt = jnp.sum((probs > pe).astype(_f32), axis=1, keepdims=True)
    eq_lo = jnp.sum(((probs == pe) & (lane < e)).astype(_f32),
                    axis=1, keepdims=True)
    w_routed = jnp.where(gt + eq_lo < TOPK, pe, 0.0)
    w = jnp.where(e >= NR, 1.0 / NS, w_routed)
    val = contrib * w

    rows = pl.ds(r * BR, BR)

    @pl.when(e == 0)
    def _init():
        out_ref[rows, :] = h_ref[...] + val

    @pl.when(e != 0)
    def _acc():
        out_ref[rows, :] += val


def kernel(x, ln1_w, ln2_w, W_kvd, W_qd, W_ku, W_qu, W_vu, W_rk, W_rq, W_o,
           sh_gate, sh_up, sh_down, rt_gate, rt_up, rt_down, W_router,
           routing_bias):
    del W_ku, W_qu
    xf = x.reshape(N, H)

    wqd_t = W_qd.T.astype(_bf16)
    wrq_t = W_rq.T.astype(_bf16)
    wrk_t = W_rk.T.astype(_bf16)
    wkvd_t = W_kvd.T.astype(_bf16)
    wvu_t = W_vu.T.astype(_bf16)
    wo_t = W_o.T.astype(_bf16)
    ln1w2 = ln1_w.reshape(1, H)
    ln2w2 = ln2_w.reshape(1, H)

    inv_freq = 1.0 / BASE ** (jnp.arange(0, H, 2, dtype=_f32) / H)
    tt = jnp.arange(T, dtype=_f32)
    freqs = tt[:, None] * inv_freq[None, :]
    emb = jnp.concatenate([freqs, freqs], axis=-1)
    cos = jnp.cos(emb) * SCALE
    sin = jnp.sin(emb) * SCALE

    wrt_t = jnp.zeros((H, RL), _f32).at[:, :NR].set(W_router.T)
    bias_p = jnp.full((1, RL), -1e30, _f32).at[0, :NR].set(routing_bias)

    all_gate_t = jnp.concatenate([rt_gate, sh_gate]).transpose(0, 2, 1).astype(_bf16)
    all_up_t = jnp.concatenate([rt_up, sh_up]).transpose(0, 2, 1).astype(_bf16)
    all_down_t = jnp.concatenate([rt_down, sh_down]).transpose(0, 2, 1).astype(_bf16)

    nrow = N // BQ
    row_spec = pl.BlockSpec((BQ, H), lambda i: (i, 0))
    full = lambda shape: pl.BlockSpec(shape, lambda i: tuple(0 for _ in shape))
    cs_spec = pl.BlockSpec((BQ, H), lambda i: (i % (T // BQ), 0))
    q, k, v = pl.pallas_call(
        _prep_kernel,
        grid=(nrow,),
        in_specs=[row_spec, full((1, H)), full((H, L)), full((L, H)),
                  full((H, H)), full((H, L)), full((L, H)), cs_spec, cs_spec],
        out_specs=[row_spec, row_spec, row_spec],
        out_shape=[jax.ShapeDtypeStruct((N, H), _bf16)] * 3,
    )(xf, ln1w2, wqd_t, wrq_t, wrk_t, wkvd_t, wvu_t, cos, sin)

    q3 = q.reshape(B, T, H)
    k3 = k.reshape(B, T, H)
    v3 = v.reshape(B, T, H)
    qb_spec = pl.BlockSpec((1, BQ, H), lambda b, i: (b, i, 0))
    kv_spec = pl.BlockSpec((1, T, H), lambda b, i: (b, 0, 0))
    y = pl.pallas_call(
        _attn_kernel,
        grid=(B, T // BQ),
        in_specs=[qb_spec, kv_spec, kv_spec],
        out_specs=qb_spec,
        out_shape=jax.ShapeDtypeStruct((B, T, H), _bf16),
    )(q3, k3, v3)

    h, xn2, logits = pl.pallas_call(
        _post_kernel,
        grid=(nrow,),
        in_specs=[row_spec, row_spec, full((H, H)), full((1, H)),
                  full((H, RL)), full((1, RL))],
        out_specs=[row_spec, row_spec, pl.BlockSpec((BQ, RL), lambda i: (i, 0))],
        out_shape=[jax.ShapeDtypeStruct((N, H), _f32),
                   jax.ShapeDtypeStruct((N, H), _bf16),
                   jax.ShapeDtypeStruct((N, RL), _f32)],
    )(y.reshape(N, H), xf, wo_t, ln2w2, wrt_t, bias_p)

    nrow_m = N // BR
    rspec = pl.BlockSpec((BR, H), lambda e, r: (r, 0))
    lspec = pl.BlockSpec((BR, RL), lambda e, r: (r, 0))
    espec_g = pl.BlockSpec((1, H, I), lambda e, r: (e, 0, 0))
    espec_d = pl.BlockSpec((1, I, H), lambda e, r: (e, 0, 0))
    out = pl.pallas_call(
        _moe_kernel,
        grid=(NE, nrow_m),
        in_specs=[rspec, rspec, lspec, espec_g, espec_g, espec_d],
        out_specs=pl.BlockSpec((N, H), lambda e, r: (0, 0)),
        out_shape=jax.ShapeDtypeStruct((N, H), _f32),
    )(xn2, h, logits, all_gate_t, all_up_t, all_down_t)

    return out.reshape(B, T, H)


# speedup vs baseline: 1.0864x; 1.0344x over previous
import jax
import jax.numpy as jnp
from jax.experimental import pallas as pl
from jax.experimental.pallas import tpu as pltpu

B, T, H = 2, 2048, 1024
L = H // 4
I = int(H * 2.0)
NS = 2
NR = 8 - NS
NE = NR + NS
TOPK = 2
BASE = 10000.0
SCALE = 1.0
EPS = 1e-5
N = B * T

BQ = 256
BR = 512
RL = 128

_f32 = jnp.float32
_bf16 = jnp.bfloat16


def _ln(x, w):
    mu = jnp.mean(x, axis=1, keepdims=True)
    xc = x - mu
    var = jnp.mean(xc * xc, axis=1, keepdims=True)
    return xc * jax.lax.rsqrt(var + EPS) * w


def _prep_kernel(x_ref, ln1w_ref, wqd_ref, wrq_ref, wrk_ref, wkvd_ref,
                 wvu_ref, cos_ref, sin_ref, q_ref, k_ref, v_ref):
    x = x_ref[...]
    xb = _ln(x, ln1w_ref[...]).astype(_bf16)
    ql = jnp.dot(xb, wqd_ref[...], preferred_element_type=_f32).astype(_bf16)
    qr = jnp.dot(ql, wrq_ref[...], preferred_element_type=_f32)
    kr = jnp.dot(xb, wrk_ref[...], preferred_element_type=_f32)
    kv = jnp.dot(xb, wkvd_ref[...], preferred_element_type=_f32).astype(_bf16)
    v = jnp.dot(kv, wvu_ref[...], preferred_element_type=_f32)
    cos = cos_ref[...]
    sin = sin_ref[...]

    def rope(t):
        t1 = t[:, :H // 2]
        t2 = t[:, H // 2:]
        rot = jnp.concatenate([-t2, t1], axis=1)
        return t * cos + rot * sin

    q_ref[...] = rope(qr).astype(_bf16)
    k_ref[...] = rope(kr).astype(_bf16)
    v_ref[...] = v.astype(_bf16)


def _attn_kernel(q_ref, k_ref, v_ref, o_ref):
    qi = pl.program_id(1)
    q = q_ref[0]
    k = k_ref[0]
    s = jax.lax.dot_general(q, k, (((1,), (1,)), ((), ())),
                            preferred_element_type=_f32) * (1.0 / 32.0)
    row = qi * BQ + jax.lax.broadcasted_iota(jnp.int32, (BQ, T), 0)
    col = jax.lax.broadcasted_iota(jnp.int32, (BQ, T), 1)
    s = jnp.where(row >= col, s, -1e30)
    m = jnp.max(s, axis=1, keepdims=True)
    p = jnp.exp(s - m)
    p = p / jnp.sum(p, axis=1, keepdims=True)
    o_ref[0] = jnp.dot(p.astype(_bf16), v_ref[0],
                       preferred_element_type=_f32).astype(_bf16)


def _post_kernel(y_ref, x_ref, wo_ref, ln2w_ref, wrt_ref, bias_ref,
                 h_ref, xn2_ref, logits_ref):
    h = x_ref[...] + jnp.dot(y_ref[...], wo_ref[...],
                             preferred_element_type=_f32)
    h_ref[...] = h
    xn2 = _ln(h, ln2w_ref[...])
    xn2_ref[...] = xn2.astype(_bf16)
    logits_ref[...] = jnp.dot(xn2, wrt_ref[...],
                              preferred_element_type=_f32) + bias_ref[...]


def _moe_kernel(xn2_ref, h_ref, logits_ref, gate_ref, up_ref, down_ref,
                out_ref):
    e = pl.program_id(1)
    xb = xn2_ref[...]
    a = jnp.dot(xb, gate_ref[0], preferred_element_type=_f32)
    b = jnp.dot(xb, up_ref[0], preferred_element_type=_f32)
    h1 = (a * jax.nn.sigmoid(a) * b).astype(_bf16)
    contrib = jnp.dot(h1, down_ref[0], preferred_element_type=_f32)

    logits = logits_ref[...]
    probs = jax.nn.sigmoid(logits)
    lane = jax.lax.broadcasted_iota(jnp.int32, (BR, RL), 1)
    pe = jnp.sum(jnp.where(lane == e, probs, 0.0), axis=1, keepdims=True)
    gt = jnp.sum((probs > pe).astype(_f32), axis=1, keepdims=True)
    eq_lo = jnp.sum(((probs == pe) & (lane < e)).astype(_f32),
                    axis=1, keepdims=True)
    w_routed = jnp.where(gt + eq_lo < TOPK, pe, 0.0)
    w = jnp.where(e >= NR, 1.0 / NS, w_routed)
    val = contrib * w

    @pl.when(e == 0)
    def _init():
        out_ref[...] = h_ref[...] + val

    @pl.when(e != 0)
    def _acc():
        out_ref[...] += val


def kernel(x, ln1_w, ln2_w, W_kvd, W_qd, W_ku, W_qu, W_vu, W_rk, W_rq, W_o,
           sh_gate, sh_up, sh_down, rt_gate, rt_up, rt_down, W_router,
           routing_bias):
    del W_ku, W_qu
    xf = x.reshape(N, H)

    wqd_t = W_qd.T.astype(_bf16)
    wrq_t = W_rq.T.astype(_bf16)
    wrk_t = W_rk.T.astype(_bf16)
    wkvd_t = W_kvd.T.astype(_bf16)
    wvu_t = W_vu.T.astype(_bf16)
    wo_t = W_o.T.astype(_bf16)
    ln1w2 = ln1_w.reshape(1, H)
    ln2w2 = ln2_w.reshape(1, H)

    inv_freq = 1.0 / BASE ** (jnp.arange(0, H, 2, dtype=_f32) / H)
    tt = jnp.arange(T, dtype=_f32)
    freqs = tt[:, None] * inv_freq[None, :]
    emb = jnp.concatenate([freqs, freqs], axis=-1)
    cos = jnp.cos(emb) * SCALE
    sin = jnp.sin(emb) * SCALE

    wrt_t = jnp.zeros((H, RL), _f32).at[:, :NR].set(W_router.T)
    bias_p = jnp.full((1, RL), -1e30, _f32).at[0, :NR].set(routing_bias)

    all_gate_t = jnp.concatenate([rt_gate, sh_gate]).transpose(0, 2, 1).astype(_bf16)
    all_up_t = jnp.concatenate([rt_up, sh_up]).transpose(0, 2, 1).astype(_bf16)
    all_down_t = jnp.concatenate([rt_down, sh_down]).transpose(0, 2, 1).astype(_bf16)

    nrow = N // BQ
    row_spec = pl.BlockSpec((BQ, H), lambda i: (i, 0))
    full = lambda shape: pl.BlockSpec(shape, lambda i: tuple(0 for _ in shape))
    cs_spec = pl.BlockSpec((BQ, H), lambda i: (i % (T // BQ), 0))
    q, k, v = pl.pallas_call(
        _prep_kernel,
        grid=(nrow,),
        in_specs=[row_spec, full((1, H)), full((H, L)), full((L, H)),
                  full((H, H)), full((H, L)), full((L, H)), cs_spec, cs_spec],
        out_specs=[row_spec, row_spec, row_spec],
        out_shape=[jax.ShapeDtypeStruct((N, H), _bf16)] * 3,
        compiler_params=pltpu.CompilerParams(
            dimension_semantics=("parallel",)),
    )(xf, ln1w2, wqd_t, wrq_t, wrk_t, wkvd_t, wvu_t, cos, sin)

    q3 = q.reshape(B, T, H)
    k3 = k.reshape(B, T, H)
    v3 = v.reshape(B, T, H)
    qb_spec = pl.BlockSpec((1, BQ, H), lambda b, i: (b, i, 0))
    kv_spec = pl.BlockSpec((1, T, H), lambda b, i: (b, 0, 0))
    y = pl.pallas_call(
        _attn_kernel,
        grid=(B, T // BQ),
        in_specs=[qb_spec, kv_spec, kv_spec],
        out_specs=qb_spec,
        out_shape=jax.ShapeDtypeStruct((B, T, H), _bf16),
        compiler_params=pltpu.CompilerParams(
            dimension_semantics=("parallel", "parallel")),
    )(q3, k3, v3)

    h, xn2, logits = pl.pallas_call(
        _post_kernel,
        grid=(nrow,),
        in_specs=[row_spec, row_spec, full((H, H)), full((1, H)),
                  full((H, RL)), full((1, RL))],
        out_specs=[row_spec, row_spec, pl.BlockSpec((BQ, RL), lambda i: (i, 0))],
        out_shape=[jax.ShapeDtypeStruct((N, H), _f32),
                   jax.ShapeDtypeStruct((N, H), _bf16),
                   jax.ShapeDtypeStruct((N, RL), _f32)],
        compiler_params=pltpu.CompilerParams(
            dimension_semantics=("parallel",)),
    )(y.reshape(N, H), xf, wo_t, ln2w2, wrt_t, bias_p)

    nrow_m = N // BR
    rspec = pl.BlockSpec((BR, H), lambda r, e: (r, 0))
    lspec = pl.BlockSpec((BR, RL), lambda r, e: (r, 0))
    espec_g = pl.BlockSpec((1, H, I), lambda r, e: (e, 0, 0))
    espec_d = pl.BlockSpec((1, I, H), lambda r, e: (e, 0, 0))
    out = pl.pallas_call(
        _moe_kernel,
        grid=(nrow_m, NE),
        in_specs=[rspec, rspec, lspec, espec_g, espec_g, espec_d],
        out_specs=pl.BlockSpec((BR, H), lambda r, e: (r, 0)),
        out_shape=jax.ShapeDtypeStruct((N, H), _f32),
        compiler_params=pltpu.CompilerParams(
            dimension_semantics=("parallel", "arbitrary")),
    )(xn2, h, logits, all_gate_t, all_up_t, all_down_t)

    return out.reshape(B, T, H)
